# Initial kernel scaffold; baseline (speedup 1.0000x reference)
import jax
import jax.numpy as jnp
from jax.experimental import pallas as pl


def kernel(sample, entity_embedding, relation_embedding):
    raise NotImplementedError("write your pallas kernel here")


# speedup vs baseline: 2.3151x; 2.3151x over previous
import functools

import jax
import jax.numpy as jnp
from jax import lax
from jax.experimental import pallas as pl
from jax.experimental.pallas import tpu as pltpu
from jax.experimental.pallas import tpu_sc as plsc

_B = 16384
_D = 128

_info = plsc.get_sparse_core_info()
_NC = _info.num_cores
_NS = _info.num_subcores
_NW = _NC * _NS
_BPW = _B // _NW
_CHUNK = 128
_NCHUNK = _BPW // _CHUNK

_mesh = plsc.VectorSubcoreMesh(core_axis_name="c", subcore_axis_name="s")


@functools.partial(
    pl.kernel,
    mesh=_mesh,
    out_type=(
        jax.ShapeDtypeStruct((_B, _D), jnp.float32),
        jax.ShapeDtypeStruct((_B, _D), jnp.float32),
        jax.ShapeDtypeStruct((_B, _D), jnp.float32),
    ),
    scratch_types=[
        pltpu.VMEM((_NCHUNK, _CHUNK), jnp.int32),
        pltpu.VMEM((_NCHUNK, _CHUNK), jnp.int32),
        pltpu.VMEM((_NCHUNK, _CHUNK), jnp.int32),
        pltpu.VMEM((_CHUNK, _D), jnp.float32),
        pltpu.VMEM((_CHUNK, _D), jnp.float32),
        pltpu.SemaphoreType.DMA,
        pltpu.SemaphoreType.DMA,
    ],
)
def _gather3(ent, rel, hidx, ridx, tidx,
             head_out, rel_out, tail_out,
             hv, rv, tv, buf0, buf1, gsem, ssem):
    wid = lax.axis_index("s") * _NC + lax.axis_index("c")
    base = wid * _BPW

    pltpu.sync_copy(hidx.at[wid], hv)
    pltpu.sync_copy(ridx.at[wid], rv)
    pltpu.sync_copy(tidx.at[wid], tv)

    tasks = [(ent, hv, head_out), (rel, rv, rel_out), (ent, tv, tail_out)]
    flat = [(t, iv, o, j) for (t, iv, o) in tasks for j in range(_NCHUNK)]
    bufs = (buf0, buf1)

    def start(i):
        t, iv, _, j = flat[i]
        return pltpu.async_copy(t.at[iv.at[j]], bufs[i % 2], gsem)

    def store(i):
        _, _, o, j = flat[i]
        return pltpu.async_copy(
            bufs[i % 2], o.at[pl.ds(base + j * _CHUNK, _CHUNK)], ssem)

    gathers = [None] * len(flat)
    stores = [None] * len(flat)
    n = len(flat)
    gathers[0] = start(0)
    for i in range(n):
        gathers[i].wait()
        if i + 1 < n:
            if i >= 1:
                stores[i - 1].wait()
            gathers[i + 1] = start(i + 1)
        stores[i] = store(i)
    stores[n - 2].wait()
    stores[n - 1].wait()


def kernel(sample, entity_embedding, relation_embedding):
    idx = sample.astype(jnp.int32).reshape(_NW, _NCHUNK, _CHUNK, 3)
    hidx = idx[..., 0]
    ridx = idx[..., 1]
    tidx = idx[..., 2]
    head, relation, tail = _gather3(
        entity_embedding, relation_embedding, hidx, ridx, tidx)
    return head, relation, tail[:, :, None]


# speedup vs baseline: 2.5865x; 1.1172x over previous
import functools

import jax
import jax.numpy as jnp
from jax import lax
from jax.experimental import pallas as pl
from jax.experimental.pallas import tpu as pltpu
from jax.experimental.pallas import tpu_sc as plsc

_B = 16384
_D = 128

_info = plsc.get_sparse_core_info()
_NC = _info.num_cores
_NS = _info.num_subcores
_NW = _NC * _NS
_BPW = _B // _NW
_CHUNK = 128
_NCHUNK = _BPW // _CHUNK
_NBUF = 4

_mesh = plsc.VectorSubcoreMesh(core_axis_name="c", subcore_axis_name="s")


@functools.partial(
    pl.kernel,
    mesh=_mesh,
    out_type=(
        jax.ShapeDtypeStruct((_B, _D), jnp.float32),
        jax.ShapeDtypeStruct((_B, _D), jnp.float32),
        jax.ShapeDtypeStruct((_B, _D), jnp.float32),
    ),
    scratch_types=[
        pltpu.VMEM((3, _NCHUNK, _CHUNK), jnp.int32),
    ] + [pltpu.VMEM((_CHUNK, _D), jnp.float32) for _ in range(_NBUF)] + [
        pltpu.SemaphoreType.DMA,
        pltpu.SemaphoreType.DMA,
    ],
)
def _gather3(ent, rel, idx, head_out, rel_out, tail_out,
             idxv, *rest):
    bufs = rest[:_NBUF]
    gsem, ssem = rest[_NBUF], rest[_NBUF + 1]

    wid = lax.axis_index("s") * _NC + lax.axis_index("c")
    base = wid * _BPW

    pltpu.sync_copy(idx.at[wid], idxv)

    tasks = [(ent, 0, head_out), (rel, 1, rel_out), (ent, 2, tail_out)]
    flat = [(t, c, o, j) for (t, c, o) in tasks for j in range(_NCHUNK)]
    n = len(flat)

    def start(i):
        t, c, _, j = flat[i]
        return pltpu.async_copy(t.at[idxv.at[c, j]], bufs[i % _NBUF], gsem)

    def store(i):
        _, _, o, j = flat[i]
        return pltpu.async_copy(
            bufs[i % _NBUF], o.at[pl.ds(base + j * _CHUNK, _CHUNK)], ssem)

    gathers = [None] * n
    stores = [None] * n
    waited = [False] * n
    for i in range(_NBUF - 1):
        gathers[i] = start(i)
    for i in range(n):
        gathers[i].wait()
        stores[i] = store(i)
        nxt = i + _NBUF - 1
        if nxt < n:
            prev = nxt - _NBUF
            if prev >= 0:
                stores[prev].wait()
                waited[prev] = True
            gathers[nxt] = start(nxt)
    for i in range(n):
        if not waited[i]:
            stores[i].wait()


def kernel(sample, entity_embedding, relation_embedding):
    idx = jnp.transpose(
        sample.astype(jnp.int32).reshape(_NW, _NCHUNK, _CHUNK, 3),
        (0, 3, 1, 2))
    head, relation, tail = _gather3(entity_embedding, relation_embedding, idx)
    return head, relation, tail[:, :, None]
